# baseline (speedup 1.0000x reference)
import functools

import jax
import jax.numpy as jnp
from jax import lax
from jax.experimental import pallas as pl
from jax.experimental.pallas import tpu as pltpu
from jax.experimental.pallas import tpu_sc as plsc

N_NODES = 10000
N_EDGES = 320000
D = 128
NUM_LAYERS = 3
BN_EPS = 1e-5

NC = 2
NS = 16
NW = NC * NS
EPT = N_EDGES // NW
K = 80
NCHUNK = EPT // K
NGRP = (NCHUNK - 1) // 4
LAST = NCHUNK - 1
RPT = 624
REM = N_NODES - NS * RPT

BR = 1000
NB = N_NODES // BR


def _seg_sum_body(x_hbm, src_hbm, dst_hbm, zrows_hbm, out_hbm,
                  r0, r1, r2, r3, si0, si1, si2, si3, di0, di1, di2, di3,
                  g0, g1, g2, g3, s0, s1, s2, s3,
                  gi0, gi1, gi2, gi3, dd0, dd1, dd2, dd3, acc_sh):
    c = lax.axis_index("c")
    s = lax.axis_index("s")
    wid = c * NS + s

    rows = (r0, r1, r2, r3)
    sidx = (si0, si1, si2, si3)
    didx = (di0, di1, di2, di3)
    gsem = (g0, g1, g2, g3)
    ssem = (s0, s1, s2, s3)
    sisem = (gi0, gi1, gi2, gi3)
    disem = (dd0, dd1, dd2, dd3)

    zoff = pl.multiple_of(s * RPT, 8)
    pltpu.sync_copy(zrows_hbm, acc_sh.at[pl.ds(zoff, RPT)])

    @pl.when(s == NS - 1)
    def _():
        pltpu.sync_copy(zrows_hbm.at[pl.ds(0, REM)],
                        acc_sh.at[pl.ds(NS * RPT, REM)])

    plsc.subcore_barrier()

    def issue_sidx(b, ch):
        pltpu.async_copy(src_hbm.at[wid, pl.ds(ch, 1)], sidx[b], sisem[b])

    def wait_sidx(b, ch):
        pltpu.make_async_copy(src_hbm.at[wid, pl.ds(ch, 1)], sidx[b],
                              sisem[b]).wait()

    def issue_didx(b, ch):
        pltpu.async_copy(dst_hbm.at[wid, pl.ds(ch, 1)], didx[b], disem[b])

    def wait_didx(b, ch):
        pltpu.make_async_copy(dst_hbm.at[wid, pl.ds(ch, 1)], didx[b],
                              disem[b]).wait()

    def issue_gather(b):
        pltpu.async_copy(x_hbm.at[sidx[b].at[0]], rows[b], gsem[b])

    def wait_gather(b):
        pltpu.make_async_copy(x_hbm.at[sidx[b].at[0]], rows[b],
                              gsem[b]).wait()

    def issue_scatter(b):
        pltpu.async_copy(rows[b], acc_sh.at[didx[b].at[0]], ssem[b],
                         add=True)

    def wait_scatter(b):
        pltpu.make_async_copy(rows[b], acc_sh.at[didx[b].at[0]],
                              ssem[b]).wait()

    def step(ch, b):
        b2 = (b + 2) % 4
        wait_gather(b)

        @pl.when(ch + 4 <= LAST)
        def _():
            issue_sidx(b, ch + 4)

        wait_didx(b, ch)
        issue_scatter(b)

        @pl.when(ch >= 2)
        def _():
            wait_scatter(b2)

        @pl.when(ch + 2 <= LAST)
        def _():
            wait_sidx(b2, ch + 2)
            issue_gather(b2)
            issue_didx(b2, ch + 2)

    for b in range(4):
        issue_sidx(b, b)
    for b in range(2):
        wait_sidx(b, b)
        issue_gather(b)
        issue_didx(b, b)

    def body(jj, carry):
        ch = jj * 4
        step(ch, 0)
        step(ch + 1, 1)
        step(ch + 2, 2)
        step(ch + 3, 3)
        return carry

    lax.fori_loop(0, NGRP, body, 0)

    wait_gather(0)
    wait_didx(0, LAST)
    issue_scatter(0)
    wait_scatter(2)
    wait_scatter(3)
    wait_scatter(0)

    plsc.subcore_barrier()

    ooff = pl.multiple_of(c * N_NODES + s * RPT, 8)
    pltpu.sync_copy(acc_sh.at[pl.ds(zoff, RPT)], out_hbm.at[pl.ds(ooff, RPT)])

    @pl.when(s == NS - 1)
    def _():
        toff = pl.multiple_of(c * N_NODES + NS * RPT, 8)
        pltpu.sync_copy(acc_sh.at[pl.ds(NS * RPT, REM)],
                        out_hbm.at[pl.ds(toff, REM)])


@functools.cache
def _seg_sum_kernel():
    return pl.kernel(
        _seg_sum_body,
        out_type=jax.ShapeDtypeStruct((2 * N_NODES, D), jnp.float32),
        mesh=plsc.VectorSubcoreMesh(core_axis_name="c", subcore_axis_name="s"),
        scratch_types=(
            [pltpu.VMEM((K, D), jnp.float32)] * 4
            + [pltpu.VMEM((1, K), jnp.int32)] * 8
            + [pltpu.SemaphoreType.DMA] * 16
            + [pltpu.VMEM_SHARED((N_NODES, D), jnp.float32)]
        ),
    )


def _layer_body(x_ref, alo_ref, ahi_ref, w1_ref, b1_ref, w2_ref, b2_ref,
                g_ref, bt_ref, y_ref, z2_scr, st_scr):
    p = pl.program_id(0)
    b = pl.program_id(1)

    @pl.when(p == 0)
    def _():
        h = x_ref[...] + alo_ref[...] + ahi_ref[...]
        z1 = jnp.maximum(
            jnp.dot(h, w1_ref[...], preferred_element_type=jnp.float32,
                    precision=lax.Precision.DEFAULT)
            + b1_ref[...], 0.0)
        z2 = (jnp.dot(z1, w2_ref[...], preferred_element_type=jnp.float32,
                      precision=lax.Precision.DEFAULT)
              + b2_ref[...])
        z2_scr[pl.ds(b * BR, BR), :] = z2
        ps = jnp.sum(z2, axis=0, keepdims=True)
        pq = jnp.sum(z2 * z2, axis=0, keepdims=True)
        blk = jnp.concatenate([ps, pq], axis=0)

        @pl.when(b == 0)
        def _():
            st_scr[...] = blk

        @pl.when(b > 0)
        def _():
            st_scr[...] = st_scr[...] + blk

    @pl.when(p == 1)
    def _():
        inv_n = 1.0 / N_NODES
        mean = st_scr[0:1, :] * inv_n
        ex2 = st_scr[1:2, :] * inv_n
        var = jnp.maximum(ex2 - mean * mean, 0.0)
        inv = lax.rsqrt(var + BN_EPS)
        z2 = z2_scr[pl.ds(b * BR, BR), :]
        y_ref[...] = jnp.maximum(
            (z2 - mean) * (inv * g_ref[...]) + bt_ref[...], 0.0)


_layer = pl.pallas_call(
    _layer_body,
    grid=(2, NB),
    in_specs=[
        pl.BlockSpec((BR, D), lambda p, b: (b * (1 - p), 0)),
        pl.BlockSpec((BR, D), lambda p, b: (b * (1 - p), 0)),
        pl.BlockSpec((BR, D), lambda p, b: (b * (1 - p) + NB, 0)),
        pl.BlockSpec((D, D), lambda p, b: (0, 0)),
        pl.BlockSpec((1, D), lambda p, b: (0, 0)),
        pl.BlockSpec((D, D), lambda p, b: (0, 0)),
        pl.BlockSpec((1, D), lambda p, b: (0, 0)),
        pl.BlockSpec((1, D), lambda p, b: (0, 0)),
        pl.BlockSpec((1, D), lambda p, b: (0, 0)),
    ],
    out_specs=pl.BlockSpec((BR, D), lambda p, b: (b * p, 0)),
    out_shape=jax.ShapeDtypeStruct((N_NODES, D), jnp.float32),
    scratch_shapes=[
        pltpu.VMEM((N_NODES, D), jnp.float32),
        pltpu.VMEM((2, D), jnp.float32),
    ],
)


def kernel(x, edge_index, batch, W1, b1, W2, b2, gamma, beta):
    del batch
    src = edge_index[0].astype(jnp.int32).reshape(NW, NCHUNK, K)
    dst = edge_index[1].astype(jnp.int32).reshape(NW, NCHUNK, K)
    zrows = jnp.zeros((RPT, D), jnp.float32)
    x = x.astype(jnp.float32)
    for i in range(NUM_LAYERS):
        part = _seg_sum_kernel()(x, src, dst, zrows)
        x = _layer(x, part, part, W1[i], b1[i].reshape(1, D),
                   W2[i], b2[i].reshape(1, D),
                   gamma[i].reshape(1, D), beta[i].reshape(1, D))
    return x


# speedup vs baseline: 1.1500x; 1.1500x over previous
import functools

import jax
import jax.numpy as jnp
from jax import lax
from jax.experimental import pallas as pl
from jax.experimental.pallas import tpu as pltpu
from jax.experimental.pallas import tpu_sc as plsc

N_NODES = 10000
N_EDGES = 320000
D = 128
NUM_LAYERS = 3
BN_EPS = 1e-5

NC = 2
NS = 16
NW = NC * NS
EPT = N_EDGES // NW
K = 80
NCHUNK = EPT // K
NTRIP = (NCHUNK - 2) // 3
RPT = 624
REM = N_NODES - NS * RPT

BR = 1000
NB = N_NODES // BR


def _seg_sum_body(x_hbm, src_hbm, dst_hbm, zrows_hbm, out_hbm,
                  src_v, rows_a, rows_b, rows_c, da, db, dc,
                  sem_a, sem_b, sem_c, sem_da, sem_db, sem_dc, acc_sh):
    c = lax.axis_index("c")
    s = lax.axis_index("s")
    wid = c * NS + s

    zoff = pl.multiple_of(s * RPT, 8)
    pltpu.sync_copy(zrows_hbm, acc_sh.at[pl.ds(zoff, RPT)])

    @pl.when(s == NS - 1)
    def _():
        pltpu.sync_copy(zrows_hbm.at[pl.ds(0, REM)],
                        acc_sh.at[pl.ds(NS * RPT, REM)])

    pltpu.sync_copy(src_hbm.at[wid], src_v)

    plsc.subcore_barrier()

    def gather(j, rows, sem):
        return pltpu.async_copy(x_hbm.at[src_v.at[j]], rows, sem)

    def didx(j, buf, sem):
        return pltpu.async_copy(dst_hbm.at[wid, pl.ds(j, 1)], buf, sem)

    def scatter(rows, buf):
        pltpu.sync_copy(rows, acc_sh.at[buf.at[0]], add=True)

    def wait_gather(j, rows, sem):
        pltpu.make_async_copy(x_hbm.at[src_v.at[j]], rows, sem).wait()

    def wait_didx(j, buf, sem):
        pltpu.make_async_copy(dst_hbm.at[wid, pl.ds(j, 1)], buf, sem).wait()

    gather(0, rows_a, sem_a)
    didx(0, da, sem_da)
    gather(1, rows_b, sem_b)
    didx(1, db, sem_db)

    def body(jj, carry):
        j = jj * 3
        gather(j + 2, rows_c, sem_c)
        didx(j + 2, dc, sem_dc)
        wait_gather(j, rows_a, sem_a)
        wait_didx(j, da, sem_da)
        scatter(rows_a, da)

        gather(j + 3, rows_a, sem_a)
        didx(j + 3, da, sem_da)
        wait_gather(j + 1, rows_b, sem_b)
        wait_didx(j + 1, db, sem_db)
        scatter(rows_b, db)

        gather(j + 4, rows_b, sem_b)
        didx(j + 4, db, sem_db)
        wait_gather(j + 2, rows_c, sem_c)
        wait_didx(j + 2, dc, sem_dc)
        scatter(rows_c, dc)
        return carry

    lax.fori_loop(0, NTRIP, body, 0)

    t0 = NCHUNK - 2
    t1 = NCHUNK - 1
    wait_gather(t0, rows_a, sem_a)
    wait_didx(t0, da, sem_da)
    scatter(rows_a, da)
    wait_gather(t1, rows_b, sem_b)
    wait_didx(t1, db, sem_db)
    scatter(rows_b, db)

    plsc.subcore_barrier()

    ooff = pl.multiple_of(c * N_NODES + s * RPT, 8)
    pltpu.sync_copy(acc_sh.at[pl.ds(zoff, RPT)], out_hbm.at[pl.ds(ooff, RPT)])

    @pl.when(s == NS - 1)
    def _():
        toff = pl.multiple_of(c * N_NODES + NS * RPT, 8)
        pltpu.sync_copy(acc_sh.at[pl.ds(NS * RPT, REM)],
                        out_hbm.at[pl.ds(toff, REM)])


@functools.cache
def _seg_sum_kernel():
    return pl.kernel(
        _seg_sum_body,
        out_type=jax.ShapeDtypeStruct((2 * N_NODES, D), jnp.float32),
        mesh=plsc.VectorSubcoreMesh(core_axis_name="c", subcore_axis_name="s"),
        scratch_types=[
            pltpu.VMEM((NCHUNK, K), jnp.int32),
            pltpu.VMEM((K, D), jnp.float32),
            pltpu.VMEM((K, D), jnp.float32),
            pltpu.VMEM((K, D), jnp.float32),
            pltpu.VMEM((1, K), jnp.int32),
            pltpu.VMEM((1, K), jnp.int32),
            pltpu.VMEM((1, K), jnp.int32),
            pltpu.SemaphoreType.DMA,
            pltpu.SemaphoreType.DMA,
            pltpu.SemaphoreType.DMA,
            pltpu.SemaphoreType.DMA,
            pltpu.SemaphoreType.DMA,
            pltpu.SemaphoreType.DMA,
            pltpu.VMEM_SHARED((N_NODES, D), jnp.float32),
        ],
    )


def _layer_body(x_ref, alo_ref, ahi_ref, w1_ref, b1_ref, w2_ref, b2_ref,
                g_ref, bt_ref, y_ref, z2_scr, st_scr):
    p = pl.program_id(0)
    b = pl.program_id(1)

    @pl.when(p == 0)
    def _():
        h = x_ref[...] + alo_ref[...] + ahi_ref[...]
        z1 = jnp.maximum(
            jnp.dot(h, w1_ref[...], preferred_element_type=jnp.float32,
                    precision=lax.Precision.DEFAULT)
            + b1_ref[...], 0.0)
        z2 = (jnp.dot(z1, w2_ref[...], preferred_element_type=jnp.float32,
                      precision=lax.Precision.DEFAULT)
              + b2_ref[...])
        z2_scr[pl.ds(b * BR, BR), :] = z2
        ps = jnp.sum(z2, axis=0, keepdims=True)
        pq = jnp.sum(z2 * z2, axis=0, keepdims=True)
        blk = jnp.concatenate([ps, pq], axis=0)

        @pl.when(b == 0)
        def _():
            st_scr[...] = blk

        @pl.when(b > 0)
        def _():
            st_scr[...] = st_scr[...] + blk

    @pl.when(p == 1)
    def _():
        inv_n = 1.0 / N_NODES
        mean = st_scr[0:1, :] * inv_n
        ex2 = st_scr[1:2, :] * inv_n
        var = jnp.maximum(ex2 - mean * mean, 0.0)
        inv = lax.rsqrt(var + BN_EPS)
        z2 = z2_scr[pl.ds(b * BR, BR), :]
        y_ref[...] = jnp.maximum(
            (z2 - mean) * (inv * g_ref[...]) + bt_ref[...], 0.0)


_layer = pl.pallas_call(
    _layer_body,
    grid=(2, NB),
    in_specs=[
        pl.BlockSpec((BR, D), lambda p, b: (b * (1 - p), 0)),
        pl.BlockSpec((BR, D), lambda p, b: (b * (1 - p), 0)),
        pl.BlockSpec((BR, D), lambda p, b: (b * (1 - p) + NB, 0)),
        pl.BlockSpec((D, D), lambda p, b: (0, 0)),
        pl.BlockSpec((1, D), lambda p, b: (0, 0)),
        pl.BlockSpec((D, D), lambda p, b: (0, 0)),
        pl.BlockSpec((1, D), lambda p, b: (0, 0)),
        pl.BlockSpec((1, D), lambda p, b: (0, 0)),
        pl.BlockSpec((1, D), lambda p, b: (0, 0)),
    ],
    out_specs=pl.BlockSpec((BR, D), lambda p, b: (b * p, 0)),
    out_shape=jax.ShapeDtypeStruct((N_NODES, D), jnp.float32),
    scratch_shapes=[
        pltpu.VMEM((N_NODES, D), jnp.float32),
        pltpu.VMEM((2, D), jnp.float32),
    ],
)


def kernel(x, edge_index, batch, W1, b1, W2, b2, gamma, beta):
    del batch
    src = edge_index[0].astype(jnp.int32).reshape(NW, NCHUNK, K)
    dst = edge_index[1].astype(jnp.int32).reshape(NW, NCHUNK, K)
    zrows = jnp.zeros((RPT, D), jnp.float32)
    x = x.astype(jnp.float32)
    for i in range(NUM_LAYERS):
        part = _seg_sum_kernel()(x, src, dst, zrows)
        x = _layer(x, part, part, W1[i], b1[i].reshape(1, D),
                   W2[i], b2[i].reshape(1, D),
                   gamma[i].reshape(1, D), beta[i].reshape(1, D))
    return x


# speedup vs baseline: 1.1911x; 1.0358x over previous
import functools

import jax
import jax.numpy as jnp
from jax import lax
from jax.experimental import pallas as pl
from jax.experimental.pallas import tpu as pltpu
from jax.experimental.pallas import tpu_sc as plsc

N_NODES = 10000
N_EDGES = 320000
D = 128
NUM_LAYERS = 3
BN_EPS = 1e-5

NC = 2
NS = 16
NW = NC * NS
EPT = N_EDGES // NW
K = 80
NCHUNK = EPT // K
NTRIP = (NCHUNK - 2) // 3
RPT = 624
REM = N_NODES - NS * RPT

BR = 2000
NB = N_NODES // BR


def _seg_sum_body(x_hbm, src_hbm, dst_hbm, zrows_hbm, out_hbm,
                  src_v, rows_a, rows_b, rows_c, da, db, dc,
                  sem_a, sem_b, sem_c, sem_da, sem_db, sem_dc, acc_sh):
    c = lax.axis_index("c")
    s = lax.axis_index("s")
    wid = c * NS + s

    zoff = pl.multiple_of(s * RPT, 8)
    pltpu.sync_copy(zrows_hbm, acc_sh.at[pl.ds(zoff, RPT)])

    @pl.when(s == NS - 1)
    def _():
        pltpu.sync_copy(zrows_hbm.at[pl.ds(0, REM)],
                        acc_sh.at[pl.ds(NS * RPT, REM)])

    pltpu.sync_copy(src_hbm.at[wid], src_v)

    plsc.subcore_barrier()

    def gather(j, rows, sem):
        return pltpu.async_copy(x_hbm.at[src_v.at[j]], rows, sem)

    def didx(j, buf, sem):
        return pltpu.async_copy(dst_hbm.at[wid, pl.ds(j, 1)], buf, sem)

    def scatter(rows, buf):
        pltpu.sync_copy(rows, acc_sh.at[buf.at[0]], add=True)

    def wait_gather(j, rows, sem):
        pltpu.make_async_copy(x_hbm.at[src_v.at[j]], rows, sem).wait()

    def wait_didx(j, buf, sem):
        pltpu.make_async_copy(dst_hbm.at[wid, pl.ds(j, 1)], buf, sem).wait()

    gather(0, rows_a, sem_a)
    didx(0, da, sem_da)
    gather(1, rows_b, sem_b)
    didx(1, db, sem_db)

    def body(jj, carry):
        j = jj * 3
        gather(j + 2, rows_c, sem_c)
        didx(j + 2, dc, sem_dc)
        wait_gather(j, rows_a, sem_a)
        wait_didx(j, da, sem_da)
        scatter(rows_a, da)

        gather(j + 3, rows_a, sem_a)
        didx(j + 3, da, sem_da)
        wait_gather(j + 1, rows_b, sem_b)
        wait_didx(j + 1, db, sem_db)
        scatter(rows_b, db)

        gather(j + 4, rows_b, sem_b)
        didx(j + 4, db, sem_db)
        wait_gather(j + 2, rows_c, sem_c)
        wait_didx(j + 2, dc, sem_dc)
        scatter(rows_c, dc)
        return carry

    lax.fori_loop(0, NTRIP, body, 0)

    t0 = NCHUNK - 2
    t1 = NCHUNK - 1
    wait_gather(t0, rows_a, sem_a)
    wait_didx(t0, da, sem_da)
    scatter(rows_a, da)
    wait_gather(t1, rows_b, sem_b)
    wait_didx(t1, db, sem_db)
    scatter(rows_b, db)

    plsc.subcore_barrier()

    ooff = pl.multiple_of(c * N_NODES + s * RPT, 8)
    pltpu.sync_copy(acc_sh.at[pl.ds(zoff, RPT)], out_hbm.at[pl.ds(ooff, RPT)])

    @pl.when(s == NS - 1)
    def _():
        toff = pl.multiple_of(c * N_NODES + NS * RPT, 8)
        pltpu.sync_copy(acc_sh.at[pl.ds(NS * RPT, REM)],
                        out_hbm.at[pl.ds(toff, REM)])


@functools.cache
def _seg_sum_kernel():
    return pl.kernel(
        _seg_sum_body,
        out_type=jax.ShapeDtypeStruct((2 * N_NODES, D), jnp.float32),
        mesh=plsc.VectorSubcoreMesh(core_axis_name="c", subcore_axis_name="s"),
        scratch_types=[
            pltpu.VMEM((NCHUNK, K), jnp.int32),
            pltpu.VMEM((K, D), jnp.float32),
            pltpu.VMEM((K, D), jnp.float32),
            pltpu.VMEM((K, D), jnp.float32),
            pltpu.VMEM((1, K), jnp.int32),
            pltpu.VMEM((1, K), jnp.int32),
            pltpu.VMEM((1, K), jnp.int32),
            pltpu.SemaphoreType.DMA,
            pltpu.SemaphoreType.DMA,
            pltpu.SemaphoreType.DMA,
            pltpu.SemaphoreType.DMA,
            pltpu.SemaphoreType.DMA,
            pltpu.SemaphoreType.DMA,
            pltpu.VMEM_SHARED((N_NODES, D), jnp.float32),
        ],
    )


def _layer_body(x_ref, alo_ref, ahi_ref, w1_ref, b1_ref, w2_ref, b2_ref,
                g_ref, bt_ref, y_ref, z2_scr, st_scr):
    p = pl.program_id(0)
    b = pl.program_id(1)

    @pl.when(p == 0)
    def _():
        h = x_ref[...] + alo_ref[...] + ahi_ref[...]
        z1 = jnp.maximum(
            jnp.dot(h, w1_ref[...], preferred_element_type=jnp.float32,
                    precision=lax.Precision.DEFAULT)
            + b1_ref[...], 0.0)
        z2 = (jnp.dot(z1, w2_ref[...], preferred_element_type=jnp.float32,
                      precision=lax.Precision.DEFAULT)
              + b2_ref[...])
        z2_scr[pl.ds(b * BR, BR), :] = z2
        ps = jnp.sum(z2, axis=0, keepdims=True)
        pq = jnp.sum(z2 * z2, axis=0, keepdims=True)
        blk = jnp.concatenate([ps, pq], axis=0)

        @pl.when(b == 0)
        def _():
            st_scr[...] = blk

        @pl.when(b > 0)
        def _():
            st_scr[...] = st_scr[...] + blk

    @pl.when(p == 1)
    def _():
        inv_n = 1.0 / N_NODES
        mean = st_scr[0:1, :] * inv_n
        ex2 = st_scr[1:2, :] * inv_n
        var = jnp.maximum(ex2 - mean * mean, 0.0)
        inv = lax.rsqrt(var + BN_EPS)
        z2 = z2_scr[pl.ds(b * BR, BR), :]
        y_ref[...] = jnp.maximum(
            (z2 - mean) * (inv * g_ref[...]) + bt_ref[...], 0.0)


_layer = pl.pallas_call(
    _layer_body,
    grid=(2, NB),
    in_specs=[
        pl.BlockSpec((BR, D), lambda p, b: (b * (1 - p), 0)),
        pl.BlockSpec((BR, D), lambda p, b: (b * (1 - p), 0)),
        pl.BlockSpec((BR, D), lambda p, b: (b * (1 - p) + NB, 0)),
        pl.BlockSpec((D, D), lambda p, b: (0, 0)),
        pl.BlockSpec((1, D), lambda p, b: (0, 0)),
        pl.BlockSpec((D, D), lambda p, b: (0, 0)),
        pl.BlockSpec((1, D), lambda p, b: (0, 0)),
        pl.BlockSpec((1, D), lambda p, b: (0, 0)),
        pl.BlockSpec((1, D), lambda p, b: (0, 0)),
    ],
    out_specs=pl.BlockSpec((BR, D), lambda p, b: (b * p, 0)),
    out_shape=jax.ShapeDtypeStruct((N_NODES, D), jnp.float32),
    scratch_shapes=[
        pltpu.VMEM((N_NODES, D), jnp.float32),
        pltpu.VMEM((2, D), jnp.float32),
    ],
)


def kernel(x, edge_index, batch, W1, b1, W2, b2, gamma, beta):
    del batch
    src = edge_index[0].astype(jnp.int32).reshape(NW, NCHUNK, K)
    dst = edge_index[1].astype(jnp.int32).reshape(NW, NCHUNK, K)
    zrows = jnp.zeros((RPT, D), jnp.float32)
    x = x.astype(jnp.float32)
    for i in range(NUM_LAYERS):
        part = _seg_sum_kernel()(x, src, dst, zrows)
        x = _layer(x, part, part, W1[i], b1[i].reshape(1, D),
                   W2[i], b2[i].reshape(1, D),
                   gamma[i].reshape(1, D), beta[i].reshape(1, D))
    return x


# speedup vs baseline: 1.1911x; 1.0000x over previous
import functools

import jax
import jax.numpy as jnp
from jax import lax
from jax.experimental import pallas as pl
from jax.experimental.pallas import tpu as pltpu
from jax.experimental.pallas import tpu_sc as plsc

N_NODES = 10000
N_EDGES = 320000
D = 128
NUM_LAYERS = 3
BN_EPS = 1e-5

NC = 2
NS = 16
NW = NC * NS
EPT = N_EDGES // NW
K = 80
NCHUNK = EPT // K
NTRIP = (NCHUNK - 2) // 3
RPT = 624
REM = N_NODES - NS * RPT

BR = 5000
NB = N_NODES // BR


def _seg_sum_body(x_hbm, src_hbm, dst_hbm, zrows_hbm, out_hbm,
                  src_v, rows_a, rows_b, rows_c, da, db, dc,
                  sem_a, sem_b, sem_c, sem_da, sem_db, sem_dc, acc_sh):
    c = lax.axis_index("c")
    s = lax.axis_index("s")
    wid = c * NS + s

    zoff = pl.multiple_of(s * RPT, 8)
    pltpu.sync_copy(zrows_hbm, acc_sh.at[pl.ds(zoff, RPT)])

    @pl.when(s == NS - 1)
    def _():
        pltpu.sync_copy(zrows_hbm.at[pl.ds(0, REM)],
                        acc_sh.at[pl.ds(NS * RPT, REM)])

    pltpu.sync_copy(src_hbm.at[wid], src_v)

    plsc.subcore_barrier()

    def gather(j, rows, sem):
        return pltpu.async_copy(x_hbm.at[src_v.at[j]], rows, sem)

    def didx(j, buf, sem):
        return pltpu.async_copy(dst_hbm.at[wid, pl.ds(j, 1)], buf, sem)

    def scatter(rows, buf):
        pltpu.sync_copy(rows, acc_sh.at[buf.at[0]], add=True)

    def wait_gather(j, rows, sem):
        pltpu.make_async_copy(x_hbm.at[src_v.at[j]], rows, sem).wait()

    def wait_didx(j, buf, sem):
        pltpu.make_async_copy(dst_hbm.at[wid, pl.ds(j, 1)], buf, sem).wait()

    gather(0, rows_a, sem_a)
    didx(0, da, sem_da)
    gather(1, rows_b, sem_b)
    didx(1, db, sem_db)

    def body(jj, carry):
        j = jj * 3
        gather(j + 2, rows_c, sem_c)
        didx(j + 2, dc, sem_dc)
        wait_gather(j, rows_a, sem_a)
        wait_didx(j, da, sem_da)
        scatter(rows_a, da)

        gather(j + 3, rows_a, sem_a)
        didx(j + 3, da, sem_da)
        wait_gather(j + 1, rows_b, sem_b)
        wait_didx(j + 1, db, sem_db)
        scatter(rows_b, db)

        gather(j + 4, rows_b, sem_b)
        didx(j + 4, db, sem_db)
        wait_gather(j + 2, rows_c, sem_c)
        wait_didx(j + 2, dc, sem_dc)
        scatter(rows_c, dc)
        return carry

    lax.fori_loop(0, NTRIP, body, 0)

    t0 = NCHUNK - 2
    t1 = NCHUNK - 1
    wait_gather(t0, rows_a, sem_a)
    wait_didx(t0, da, sem_da)
    scatter(rows_a, da)
    wait_gather(t1, rows_b, sem_b)
    wait_didx(t1, db, sem_db)
    scatter(rows_b, db)

    plsc.subcore_barrier()

    ooff = pl.multiple_of(c * N_NODES + s * RPT, 8)
    pltpu.sync_copy(acc_sh.at[pl.ds(zoff, RPT)], out_hbm.at[pl.ds(ooff, RPT)])

    @pl.when(s == NS - 1)
    def _():
        toff = pl.multiple_of(c * N_NODES + NS * RPT, 8)
        pltpu.sync_copy(acc_sh.at[pl.ds(NS * RPT, REM)],
                        out_hbm.at[pl.ds(toff, REM)])


@functools.cache
def _seg_sum_kernel():
    return pl.kernel(
        _seg_sum_body,
        out_type=jax.ShapeDtypeStruct((2 * N_NODES, D), jnp.float32),
        mesh=plsc.VectorSubcoreMesh(core_axis_name="c", subcore_axis_name="s"),
        scratch_types=[
            pltpu.VMEM((NCHUNK, K), jnp.int32),
            pltpu.VMEM((K, D), jnp.float32),
            pltpu.VMEM((K, D), jnp.float32),
            pltpu.VMEM((K, D), jnp.float32),
            pltpu.VMEM((1, K), jnp.int32),
            pltpu.VMEM((1, K), jnp.int32),
            pltpu.VMEM((1, K), jnp.int32),
            pltpu.SemaphoreType.DMA,
            pltpu.SemaphoreType.DMA,
            pltpu.SemaphoreType.DMA,
            pltpu.SemaphoreType.DMA,
            pltpu.SemaphoreType.DMA,
            pltpu.SemaphoreType.DMA,
            pltpu.VMEM_SHARED((N_NODES, D), jnp.float32),
        ],
    )


def _layer_body(x_ref, alo_ref, ahi_ref, w1_ref, b1_ref, w2_ref, b2_ref,
                g_ref, bt_ref, y_ref, z2_scr, st_scr):
    p = pl.program_id(0)
    b = pl.program_id(1)

    @pl.when(p == 0)
    def _():
        h = x_ref[...] + alo_ref[...] + ahi_ref[...]
        z1 = jnp.maximum(
            jnp.dot(h, w1_ref[...], preferred_element_type=jnp.float32,
                    precision=lax.Precision.DEFAULT)
            + b1_ref[...], 0.0)
        z2 = (jnp.dot(z1, w2_ref[...], preferred_element_type=jnp.float32,
                      precision=lax.Precision.DEFAULT)
              + b2_ref[...])
        z2_scr[pl.ds(b * BR, BR), :] = z2
        ps = jnp.sum(z2, axis=0, keepdims=True)
        pq = jnp.sum(z2 * z2, axis=0, keepdims=True)
        blk = jnp.concatenate([ps, pq], axis=0)

        @pl.when(b == 0)
        def _():
            st_scr[...] = blk

        @pl.when(b > 0)
        def _():
            st_scr[...] = st_scr[...] + blk

    @pl.when(p == 1)
    def _():
        inv_n = 1.0 / N_NODES
        mean = st_scr[0:1, :] * inv_n
        ex2 = st_scr[1:2, :] * inv_n
        var = jnp.maximum(ex2 - mean * mean, 0.0)
        inv = lax.rsqrt(var + BN_EPS)
        z2 = z2_scr[pl.ds(b * BR, BR), :]
        y_ref[...] = jnp.maximum(
            (z2 - mean) * (inv * g_ref[...]) + bt_ref[...], 0.0)


_layer = pl.pallas_call(
    _layer_body,
    grid=(2, NB),
    in_specs=[
        pl.BlockSpec((BR, D), lambda p, b: (b * (1 - p), 0)),
        pl.BlockSpec((BR, D), lambda p, b: (b * (1 - p), 0)),
        pl.BlockSpec((BR, D), lambda p, b: (b * (1 - p) + NB, 0)),
        pl.BlockSpec((D, D), lambda p, b: (0, 0)),
        pl.BlockSpec((1, D), lambda p, b: (0, 0)),
        pl.BlockSpec((D, D), lambda p, b: (0, 0)),
        pl.BlockSpec((1, D), lambda p, b: (0, 0)),
        pl.BlockSpec((1, D), lambda p, b: (0, 0)),
        pl.BlockSpec((1, D), lambda p, b: (0, 0)),
    ],
    out_specs=pl.BlockSpec((BR, D), lambda p, b: (b * p, 0)),
    out_shape=jax.ShapeDtypeStruct((N_NODES, D), jnp.float32),
    scratch_shapes=[
        pltpu.VMEM((N_NODES, D), jnp.float32),
        pltpu.VMEM((2, D), jnp.float32),
    ],
)


def kernel(x, edge_index, batch, W1, b1, W2, b2, gamma, beta):
    del batch
    src = edge_index[0].astype(jnp.int32).reshape(NW, NCHUNK, K)
    dst = edge_index[1].astype(jnp.int32).reshape(NW, NCHUNK, K)
    zrows = jnp.zeros((RPT, D), jnp.float32)
    x = x.astype(jnp.float32)
    for i in range(NUM_LAYERS):
        part = _seg_sum_kernel()(x, src, dst, zrows)
        x = _layer(x, part, part, W1[i], b1[i].reshape(1, D),
                   W2[i], b2[i].reshape(1, D),
                   gamma[i].reshape(1, D), beta[i].reshape(1, D))
    return x
